# baseline (speedup 1.0000x reference)
import functools

import jax
import jax.numpy as jnp
from jax import lax
from jax.experimental import pallas as pl
from jax.experimental.pallas import tpu as pltpu
from jax.experimental.pallas import tpu_sc as plsc

B, L = 16384, 50
V, D = 100000, 128
H = 4096
NA, NS = 12, 3

NC, NSC = 2, 16
NW = NC * NSC
CB = 2
RPC = CB * L
NBUF = 3
OROWS = 128
FMASK = OROWS // CB - 1

NSLICE = 4


def _make_sc_body(bs):
    bpw = bs // NW
    nch = bpw // CB

    def _sc_pool_body(x_hbm, emb_hbm, avg_hbm, idx_v, rows_v, out_v,
                      sem0, sem1, sem2):
        wid = lax.axis_index("s") * NC + lax.axis_index("c")
        base = pl.multiple_of(wid * bpw, bpw)
        sems = [sem0, sem1, sem2]

        pltpu.sync_copy(
            x_hbm.at[pl.ds(pl.multiple_of(wid * nch, nch), nch), :], idx_v)

        def fire(g, buf):
            pltpu.async_copy(emb_hbm.at[idx_v.at[g]], rows_v.at[buf],
                             sems[buf])

        def drain(buf):
            pltpu.make_async_copy(
                emb_hbm.at[idx_v.at[0]], rows_v.at[buf], sems[buf]).wait()

        U = 10

        def accum_chunk(buf, g):
            orow = (CB * g) & (OROWS - 1)
            for r in range(CB):
                def acc_step(j, accs):
                    row = r * L + j * U
                    new = list(accs)
                    for u in range(U):
                        for c in range(D // 16):
                            new[c] = new[c] + rows_v[buf, row + u,
                                                     pl.ds(c * 16, 16)]
                    return tuple(new)

                accs = lax.fori_loop(
                    0, L // U, acc_step,
                    tuple(jnp.zeros((16,), jnp.float32)
                          for _ in range(D // 16)))
                for c in range(D // 16):
                    out_v[orow + r, pl.ds(c * 16, 16)] = accs[c]

        def flush(g):
            row0 = pl.multiple_of(base + CB * g - (OROWS - CB), OROWS)
            pltpu.sync_copy(out_v, avg_hbm.at[pl.ds(row0, OROWS), :])

        for buf in range(NBUF):
            fire(buf, buf)

        def body(h, carry):
            for u in range(NBUF):
                g = NBUF * h + u
                drain(u)
                accum_chunk(u, g)

                @pl.when(g + NBUF < nch)
                def _():
                    fire(g + NBUF, u)

                @pl.when((g & FMASK) == FMASK)
                def _():
                    flush(g)
            return carry

        lax.fori_loop(0, nch // NBUF, body, 0)
        for g in range(NBUF * (nch // NBUF), nch):
            buf = g % NBUF
            drain(buf)
            accum_chunk(buf, g)
            if (g & FMASK) == FMASK:
                flush(g)

    return _sc_pool_body, bpw, nch


def _sc_pool(x_chunks, embedding):
    bs = x_chunks.shape[0] * CB
    body, bpw, nch = _make_sc_body(bs)
    mesh = plsc.VectorSubcoreMesh(core_axis_name="c", subcore_axis_name="s")
    f = functools.partial(
        pl.kernel,
        mesh=mesh,
        out_type=jax.ShapeDtypeStruct((bs, D), jnp.float32),
        scratch_types=[
            pltpu.VMEM((nch, RPC), jnp.int32),
            pltpu.VMEM((NBUF, RPC, D), jnp.float32),
            pltpu.VMEM((OROWS, D), jnp.float32),
            pltpu.SemaphoreType.DMA,
            pltpu.SemaphoreType.DMA,
            pltpu.SemaphoreType.DMA,
        ],
    )(body)
    return f(x_chunks, embedding)


def _mlp_body(sum_ref, aid_ref, x_ref, aemb_ref, w1a_ref, w1b_ref, b1_ref,
              w2_ref, b2_ref, w3_ref, b3_ref, out_ref):
    cnt = jnp.sum((x_ref[...] != 0).astype(jnp.float32), axis=1, keepdims=True)
    avg = (sum_ref[...] / jnp.maximum(cnt, 1.0)).astype(jnp.bfloat16)
    bm = aid_ref.shape[0]
    onehot = (aid_ref[...] ==
              lax.broadcasted_iota(jnp.int32, (bm, NA), 1)
              ).astype(jnp.bfloat16)
    asp = jnp.dot(onehot, aemb_ref[...],
                  preferred_element_type=jnp.float32).astype(jnp.bfloat16)
    h1 = jnp.dot(avg, w1a_ref[...], preferred_element_type=jnp.float32)
    h1 = h1 + jnp.dot(asp, w1b_ref[...], preferred_element_type=jnp.float32)
    h1 = jnp.maximum(h1 + b1_ref[...], 0.0).astype(jnp.bfloat16)
    h2 = jnp.dot(h1, w2_ref[...], preferred_element_type=jnp.float32)
    h2 = jnp.maximum(h2 + b2_ref[...], 0.0).astype(jnp.bfloat16)
    out = jnp.dot(h2, w3_ref[...], preferred_element_type=jnp.float32)
    out_ref[...] = out + b3_ref[...]


def _mlp(emb_sum, aspect_ids, x, aspect_embedding, W1a, W1b, b1, W2, b2,
         W3, b3):
    bs = emb_sum.shape[0]
    BM = 1024
    grid = (bs // BM,)
    return pl.pallas_call(
        _mlp_body,
        grid=grid,
        in_specs=[
            pl.BlockSpec((BM, D), lambda i: (i, 0)),
            pl.BlockSpec((BM, 1), lambda i: (i, 0)),
            pl.BlockSpec((BM, L), lambda i: (i, 0)),
            pl.BlockSpec((NA, D), lambda i: (0, 0)),
            pl.BlockSpec((D, H), lambda i: (0, 0)),
            pl.BlockSpec((D, H), lambda i: (0, 0)),
            pl.BlockSpec((1, H), lambda i: (0, 0)),
            pl.BlockSpec((H, H // 2), lambda i: (0, 0)),
            pl.BlockSpec((1, H // 2), lambda i: (0, 0)),
            pl.BlockSpec((H // 2, NS), lambda i: (0, 0)),
            pl.BlockSpec((1, NS), lambda i: (0, 0)),
        ],
        out_specs=pl.BlockSpec((BM, NS), lambda i: (i, 0)),
        out_shape=jax.ShapeDtypeStruct((bs, NS), jnp.float32),
    )(emb_sum, aspect_ids.reshape(bs, 1), x, aspect_embedding, W1a, W1b,
      b1.reshape(1, H), W2, b2.reshape(1, H // 2), W3, b3.reshape(1, NS))


def kernel(x, aspect_ids, embedding, aspect_embedding, W1, b1, W2, b2, W3, b3):
    bs = B // NSLICE
    W1a = W1[:D].astype(jnp.bfloat16)
    W1b = W1[D:].astype(jnp.bfloat16)
    W2 = W2.astype(jnp.bfloat16)
    W3 = W3.astype(jnp.bfloat16)
    aspect_embedding = aspect_embedding.astype(jnp.bfloat16)
    sums = [
        _sc_pool(x[i * bs:(i + 1) * bs].reshape(bs // CB, RPC), embedding)
        for i in range(NSLICE)
    ]
    outs = [
        _mlp(sums[i], aspect_ids[i * bs:(i + 1) * bs],
             x[i * bs:(i + 1) * bs], aspect_embedding, W1a, W1b, b1, W2, b2,
             W3, b3)
        for i in range(NSLICE)
    ]
    return jnp.concatenate(outs, axis=0)


# speedup vs baseline: 1.0709x; 1.0709x over previous
import functools

import jax
import jax.numpy as jnp
from jax import lax
from jax.experimental import pallas as pl
from jax.experimental.pallas import tpu as pltpu
from jax.experimental.pallas import tpu_sc as plsc

B, L = 16384, 50
V, D = 100000, 128
H = 4096
NA, NS = 12, 3

NC, NSC = 2, 16
NW = NC * NSC
CB = 2
RPC = CB * L
NBUF = 3
OROWS = 128
FMASK = OROWS // CB - 1

NSLICE = 4


def _make_sc_body(bs):
    bpw = bs // NW
    nch = bpw // CB

    def _sc_pool_body(x_hbm, emb_hbm, avg_hbm, idx_v, rows_v, out_v,
                      sem0, sem1, sem2):
        wid = lax.axis_index("s") * NC + lax.axis_index("c")
        base = pl.multiple_of(wid * bpw, bpw)
        sems = [sem0, sem1, sem2]

        pltpu.sync_copy(
            x_hbm.at[pl.ds(pl.multiple_of(wid * nch, nch), nch), :], idx_v)

        def fire(g, buf):
            pltpu.async_copy(emb_hbm.at[idx_v.at[g]], rows_v.at[buf],
                             sems[buf])

        def drain(buf):
            pltpu.make_async_copy(
                emb_hbm.at[idx_v.at[0]], rows_v.at[buf], sems[buf]).wait()

        U = 10

        def accum_chunk(buf, g):
            orow = (CB * g) & (OROWS - 1)
            for r in range(CB):
                def acc_step(j, accs):
                    row = r * L + j * U
                    new = list(accs)
                    for u in range(U):
                        for c in range(D // 16):
                            new[c] = new[c] + rows_v[buf, row + u,
                                                     pl.ds(c * 16, 16)]
                    return tuple(new)

                accs = lax.fori_loop(
                    0, L // U, acc_step,
                    tuple(jnp.zeros((16,), jnp.float32)
                          for _ in range(D // 16)))
                for c in range(D // 16):
                    out_v[orow + r, pl.ds(c * 16, 16)] = accs[c]

        def flush(g):
            row0 = pl.multiple_of(base + CB * g - (OROWS - CB), OROWS)
            pltpu.sync_copy(out_v, avg_hbm.at[pl.ds(row0, OROWS), :])

        for buf in range(NBUF):
            fire(buf, buf)

        def body(h, carry):
            for u in range(NBUF):
                g = NBUF * h + u
                drain(u)
                accum_chunk(u, g)

                @pl.when(g + NBUF < nch)
                def _():
                    fire(g + NBUF, u)

                @pl.when((g & FMASK) == FMASK)
                def _():
                    flush(g)
            return carry

        lax.fori_loop(0, nch // NBUF, body, 0)
        for g in range(NBUF * (nch // NBUF), nch):
            buf = g % NBUF
            drain(buf)
            accum_chunk(buf, g)
            if (g & FMASK) == FMASK:
                flush(g)

    return _sc_pool_body, bpw, nch


def _sc_pool(x_chunks, embedding):
    bs = x_chunks.shape[0] * CB
    body, bpw, nch = _make_sc_body(bs)
    mesh = plsc.VectorSubcoreMesh(core_axis_name="c", subcore_axis_name="s")
    f = functools.partial(
        pl.kernel,
        mesh=mesh,
        out_type=jax.ShapeDtypeStruct((bs, D), jnp.float32),
        scratch_types=[
            pltpu.VMEM((nch, RPC), jnp.int32),
            pltpu.VMEM((NBUF, RPC, D), jnp.float32),
            pltpu.VMEM((OROWS, D), jnp.float32),
            pltpu.SemaphoreType.DMA,
            pltpu.SemaphoreType.DMA,
            pltpu.SemaphoreType.DMA,
        ],
    )(body)
    return f(x_chunks, embedding)


def _mlp_body(sum_ref, aid_ref, x_ref, aemb_ref, w1_ref, b1_ref,
              w2_ref, b2_ref, w3_ref, b3_ref, out_ref):
    cnt = jnp.sum((x_ref[...] != 0).astype(jnp.float32), axis=1, keepdims=True)
    avg = (sum_ref[...] / jnp.maximum(cnt, 1.0)).astype(jnp.bfloat16)
    bm = aid_ref.shape[0]
    onehot = (aid_ref[...] ==
              lax.broadcasted_iota(jnp.int32, (bm, NA), 1)
              ).astype(jnp.bfloat16)
    asp = jnp.dot(onehot, aemb_ref[...],
                  preferred_element_type=jnp.float32).astype(jnp.bfloat16)
    xcat = jnp.concatenate([avg, asp], axis=1)
    h1 = jnp.dot(xcat, w1_ref[...], preferred_element_type=jnp.float32)
    h1 = jnp.maximum(h1 + b1_ref[...], 0.0).astype(jnp.bfloat16)
    h2 = jnp.dot(h1, w2_ref[...], preferred_element_type=jnp.float32)
    h2 = jnp.maximum(h2 + b2_ref[...], 0.0).astype(jnp.bfloat16)
    out = jnp.dot(h2, w3_ref[...], preferred_element_type=jnp.float32)
    out_ref[...] = out + b3_ref[...]


def _mlp(emb_sum, aspect_ids, x, aspect_embedding, W1, b1, W2, b2, W3, b3):
    bs = emb_sum.shape[0]
    BM = 512
    grid = (bs // BM,)
    return pl.pallas_call(
        _mlp_body,
        grid=grid,
        in_specs=[
            pl.BlockSpec((BM, D), lambda i: (i, 0)),
            pl.BlockSpec((BM, 1), lambda i: (i, 0)),
            pl.BlockSpec((BM, L), lambda i: (i, 0)),
            pl.BlockSpec((NA, D), lambda i: (0, 0)),
            pl.BlockSpec((2 * D, H), lambda i: (0, 0)),
            pl.BlockSpec((1, H), lambda i: (0, 0)),
            pl.BlockSpec((H, H // 2), lambda i: (0, 0)),
            pl.BlockSpec((1, H // 2), lambda i: (0, 0)),
            pl.BlockSpec((H // 2, NS), lambda i: (0, 0)),
            pl.BlockSpec((1, NS), lambda i: (0, 0)),
        ],
        out_specs=pl.BlockSpec((BM, NS), lambda i: (i, 0)),
        out_shape=jax.ShapeDtypeStruct((bs, NS), jnp.float32),
    )(emb_sum, aspect_ids.reshape(bs, 1), x, aspect_embedding, W1,
      b1.reshape(1, H), W2, b2.reshape(1, H // 2), W3, b3.reshape(1, NS))


def kernel(x, aspect_ids, embedding, aspect_embedding, W1, b1, W2, b2, W3, b3):
    bs = B // NSLICE
    W1 = W1.astype(jnp.bfloat16)
    W2 = W2.astype(jnp.bfloat16)
    W3 = W3.astype(jnp.bfloat16)
    aspect_embedding = aspect_embedding.astype(jnp.bfloat16)
    sums = [
        _sc_pool(x[i * bs:(i + 1) * bs].reshape(bs // CB, RPC), embedding)
        for i in range(NSLICE)
    ]
    outs = [
        _mlp(sums[i], aspect_ids[i * bs:(i + 1) * bs],
             x[i * bs:(i + 1) * bs], aspect_embedding, W1, b1, W2, b2,
             W3, b3)
        for i in range(NSLICE)
    ]
    return jnp.concatenate(outs, axis=0)


# speedup vs baseline: 1.0757x; 1.0045x over previous
import functools

import jax
import jax.numpy as jnp
from jax import lax
from jax.experimental import pallas as pl
from jax.experimental.pallas import tpu as pltpu
from jax.experimental.pallas import tpu_sc as plsc

B, L = 16384, 50
V, D = 100000, 128
H = 4096
NA, NS = 12, 3

NC, NSC = 2, 16
NW = NC * NSC
CB = 2
RPC = CB * L
NBUF = 3
OROWS = 128
FMASK = OROWS // CB - 1

NSLICE = 4


NACC = 4


def _make_sc_body(bs):
    bpw = bs // NW

    def _sc_pool_body(xt_hbm, emb_hbm, avg_hbm, idx_v, acc_v,
                      sem0, sem1, sem2, sem3):
        wid = lax.axis_index("s") * NC + lax.axis_index("c")
        base = pl.multiple_of(wid * bpw, bpw)
        sems = [sem0, sem1, sem2, sem3]

        pltpu.sync_copy(xt_hbm.at[:, pl.ds(base, bpw)], idx_v)

        zeros = jnp.zeros((16,), jnp.float32)

        def zero_step(r, carry):
            for a in range(NACC):
                for c in range(D // 16):
                    acc_v[a, r, pl.ds(c * 16, 16)] = zeros
            return carry

        lax.fori_loop(0, bpw, zero_step, 0)

        def fire(j, a):
            pltpu.async_copy(emb_hbm.at[idx_v.at[j]], acc_v.at[a],
                             sems[a], add=True)

        def drain(a):
            pltpu.make_async_copy(
                emb_hbm.at[idx_v.at[0]], acc_v.at[a], sems[a]).wait()

        for j in range(NACC):
            fire(j, j)
        for j in range(NACC, L):
            a = j % NACC
            drain(a)
            fire(j, a)
        for a in range(L % NACC, L % NACC + NACC):
            drain(a % NACC)

        def fold_step(r, carry):
            for c in range(D // 16):
                s = acc_v[0, r, pl.ds(c * 16, 16)]
                for a in range(1, NACC):
                    s = s + acc_v[a, r, pl.ds(c * 16, 16)]
                acc_v[0, r, pl.ds(c * 16, 16)] = s
            return carry

        lax.fori_loop(0, bpw, fold_step, 0)
        pltpu.sync_copy(acc_v.at[0], avg_hbm.at[pl.ds(base, bpw), :])

    return _sc_pool_body, bpw


def _sc_pool(xt, embedding):
    bs = xt.shape[1]
    body, bpw = _make_sc_body(bs)
    mesh = plsc.VectorSubcoreMesh(core_axis_name="c", subcore_axis_name="s")
    f = functools.partial(
        pl.kernel,
        mesh=mesh,
        out_type=jax.ShapeDtypeStruct((bs, D), jnp.float32),
        scratch_types=[
            pltpu.VMEM((L, bpw), jnp.int32),
            pltpu.VMEM((NACC, bpw, D), jnp.float32),
            pltpu.SemaphoreType.DMA,
            pltpu.SemaphoreType.DMA,
            pltpu.SemaphoreType.DMA,
            pltpu.SemaphoreType.DMA,
        ],
    )(body)
    return f(xt, embedding)


def _mlp_body(sum_ref, aid_ref, x_ref, aemb_ref, w1_ref, b1_ref,
              w2_ref, b2_ref, w3_ref, b3_ref, out_ref):
    cnt = jnp.sum((x_ref[...] != 0).astype(jnp.float32), axis=1, keepdims=True)
    avg = (sum_ref[...] / jnp.maximum(cnt, 1.0)).astype(jnp.bfloat16)
    bm = aid_ref.shape[0]
    onehot = (aid_ref[...] ==
              lax.broadcasted_iota(jnp.int32, (bm, NA), 1)
              ).astype(jnp.bfloat16)
    asp = jnp.dot(onehot, aemb_ref[...],
                  preferred_element_type=jnp.float32).astype(jnp.bfloat16)
    xcat = jnp.concatenate([avg, asp], axis=1)
    h1 = jnp.dot(xcat, w1_ref[...], preferred_element_type=jnp.float32)
    h1 = jnp.maximum(h1 + b1_ref[...], 0.0).astype(jnp.bfloat16)
    h2 = jnp.dot(h1, w2_ref[...], preferred_element_type=jnp.float32)
    h2 = jnp.maximum(h2 + b2_ref[...], 0.0).astype(jnp.bfloat16)
    out = jnp.dot(h2, w3_ref[...], preferred_element_type=jnp.float32)
    out_ref[...] = out + b3_ref[...]


def _mlp(emb_sum, aspect_ids, x, aspect_embedding, W1, b1, W2, b2, W3, b3):
    bs = emb_sum.shape[0]
    BM = 512
    grid = (bs // BM,)
    return pl.pallas_call(
        _mlp_body,
        grid=grid,
        in_specs=[
            pl.BlockSpec((BM, D), lambda i: (i, 0)),
            pl.BlockSpec((BM, 1), lambda i: (i, 0)),
            pl.BlockSpec((BM, L), lambda i: (i, 0)),
            pl.BlockSpec((NA, D), lambda i: (0, 0)),
            pl.BlockSpec((2 * D, H), lambda i: (0, 0)),
            pl.BlockSpec((1, H), lambda i: (0, 0)),
            pl.BlockSpec((H, H // 2), lambda i: (0, 0)),
            pl.BlockSpec((1, H // 2), lambda i: (0, 0)),
            pl.BlockSpec((H // 2, NS), lambda i: (0, 0)),
            pl.BlockSpec((1, NS), lambda i: (0, 0)),
        ],
        out_specs=pl.BlockSpec((BM, NS), lambda i: (i, 0)),
        out_shape=jax.ShapeDtypeStruct((bs, NS), jnp.float32),
    )(emb_sum, aspect_ids.reshape(bs, 1), x, aspect_embedding, W1,
      b1.reshape(1, H), W2, b2.reshape(1, H // 2), W3, b3.reshape(1, NS))


def kernel(x, aspect_ids, embedding, aspect_embedding, W1, b1, W2, b2, W3, b3):
    bs = B // NSLICE
    W1 = W1.astype(jnp.bfloat16)
    W2 = W2.astype(jnp.bfloat16)
    W3 = W3.astype(jnp.bfloat16)
    aspect_embedding = aspect_embedding.astype(jnp.bfloat16)
    xt = x.T
    sums = [
        _sc_pool(xt[:, i * bs:(i + 1) * bs], embedding)
        for i in range(NSLICE)
    ]
    outs = [
        _mlp(sums[i], aspect_ids[i * bs:(i + 1) * bs],
             x[i * bs:(i + 1) * bs], aspect_embedding, W1, b1, W2, b2,
             W3, b3)
        for i in range(NSLICE)
    ]
    return jnp.concatenate(outs, axis=0)
